# Initial kernel scaffold; baseline (speedup 1.0000x reference)
import jax
import jax.numpy as jnp
from jax.experimental import pallas as pl


def kernel(in_feat, edge_index, W0, b0, W1, b1, W2, b2, W3, b3):
    raise NotImplementedError("write your pallas kernel here")


# speedup vs baseline: 12.7650x; 12.7650x over previous
import functools

import jax
import jax.numpy as jnp
from jax import lax
from jax.experimental import pallas as pl
from jax.experimental.pallas import tpu as pltpu
from jax.experimental.pallas import tpu_sc as plsc

N = 100000
NPAD = 100096
E = 3200000
EPAD = 3276800
NC, NS = 2, 16
NW = NC * NS
CH = 2048
ROWS = CH // 128
TPS = NPAD // NS
EPT = EPAD // NW
NCHUNK = EPT // CH

_mesh = plsc.VectorSubcoreMesh(core_axis_name="c", subcore_axis_name="s")


def _f32(shape):
    return jax.ShapeDtypeStruct(shape, jnp.float32)


@functools.partial(
    pl.kernel,
    out_type=_f32((NC * NPAD,)),
    mesh=_mesh,
    compiler_params=pltpu.CompilerParams(needs_layout_passes=False, use_tc_tiling_on_sc=False),
    scratch_types=dict(
        xbuf=pltpu.VMEM((NPAD,), jnp.float32),
        srcb=pltpu.VMEM((CH,), jnp.int32),
        dstb=pltpu.VMEM((ROWS, 128), jnp.int32),
        valb=pltpu.VMEM((ROWS, 128), jnp.float32),
        zbuf=pltpu.VMEM((TPS,), jnp.float32),
        acc=pltpu.VMEM_SHARED((NPAD,), jnp.float32),
    ),
)
def _f1_scalar_scatter(src_h, dst2_h, x_h, out_h, *, xbuf, srcb, dstb,
                       valb, zbuf, acc):
    c = lax.axis_index("c")
    s = lax.axis_index("s")
    wid = c * NS + s
    base = wid * EPT
    baserow = base // 128
    def zfill(i, carry):
        zbuf[pl.ds(i * 16, 16)] = jnp.zeros((16,), jnp.float32)
        return carry
    lax.fori_loop(0, TPS // 16, zfill, 0)
    off = pl.multiple_of(s * TPS, 8)
    pltpu.sync_copy(zbuf, acc.at[pl.ds(off, TPS)])
    pltpu.sync_copy(x_h, xbuf)
    plsc.subcore_barrier()

    def chunk(ci, carry):
        pltpu.sync_copy(src_h.at[pl.ds(pl.multiple_of(base + ci * CH, 128), CH)], srcb)
        row0 = pl.multiple_of(baserow + ci * ROWS, 8)
        pltpu.sync_copy(dst2_h.at[pl.ds(row0, ROWS), :], dstb)
        for r in range(ROWS):
            for k in range(8):
                sv = srcb[pl.ds(r * 128 + k * 16, 16)]
                valb[r, pl.ds(k * 16, 16)] = plsc.load_gather(xbuf, [sv])
        for r in range(ROWS):
            pltpu.sync_copy(valb.at[r], acc.at[dstb.at[r]], add=True)
        return carry

    lax.fori_loop(0, NCHUNK, chunk, 0)
    plsc.subcore_barrier()
    off = pl.multiple_of(s * TPS, 8)
    oof = pl.multiple_of(c * NPAD + s * TPS, 8)
    pltpu.sync_copy(acc.at[pl.ds(off, TPS)], zbuf)
    pltpu.sync_copy(zbuf, out_h.at[pl.ds(oof, TPS)])


EPT_B = EPAD // NS
NCHUNK_B = EPT_B // CH

@functools.partial(
    pl.kernel,
    out_type=_f32((NC * 8 * NPAD,)),
    mesh=_mesh,
    compiler_params=pltpu.CompilerParams(needs_layout_passes=False, use_tc_tiling_on_sc=False),
    scratch_types=dict(
        srcb=pltpu.VMEM((ROWS, 128), jnp.int32),
        dstb=pltpu.VMEM((ROWS, 128), jnp.int32),
        vbs=[pltpu.VMEM((128,), jnp.float32) for _ in range(8)],
        zbuf=pltpu.VMEM((TPS,), jnp.float32),
        sem=pltpu.SemaphoreType.DMA,
        planes=[pltpu.VMEM_SHARED((NPAD,), jnp.float32) for _ in range(8)],
        accs=[pltpu.VMEM_SHARED((NPAD,), jnp.float32) for _ in range(8)],
    ),
)
def _f_row_scatter(src2_h, dst2_h, tflat_h, out_h, *, srcb, dstb, vbs, zbuf,
                   sem, planes, accs):
    c = lax.axis_index("c")
    s = lax.axis_index("s")
    base = s * EPT_B
    baserow = base // 128
    off = pl.multiple_of(s * TPS, 8)
    def zfill(i, carry):
        zbuf[pl.ds(i * 16, 16)] = jnp.zeros((16,), jnp.float32)
        return carry
    lax.fori_loop(0, TPS // 16, zfill, 0)
    for i in range(8):
        pltpu.sync_copy(zbuf, accs[i].at[pl.ds(off, TPS)])
    for i in range(8):
        hoff = pl.multiple_of((c * 8 + i) * NPAD + s * TPS, 8)
        pltpu.sync_copy(tflat_h.at[pl.ds(hoff, TPS)], zbuf)
        pltpu.sync_copy(zbuf, planes[i].at[pl.ds(off, TPS)])
    plsc.subcore_barrier()

    def chunk(ci, carry):
        row0 = pl.multiple_of(baserow + ci * ROWS, 8)
        pltpu.sync_copy(src2_h.at[pl.ds(row0, ROWS), :], srcb)
        pltpu.sync_copy(dst2_h.at[pl.ds(row0, ROWS), :], dstb)
        for r in range(ROWS):
            gs = [pltpu.async_copy(planes[i].at[srcb.at[r]], vbs[i], sem)
                  for i in range(8)]
            for g in gs:
                g.wait()
            for i in range(8):
                pltpu.sync_copy(vbs[i], accs[i].at[dstb.at[r]], add=True)
        return carry

    lax.fori_loop(0, NCHUNK_B, chunk, 0)
    plsc.subcore_barrier()
    for i in range(8):
        ooff = pl.multiple_of((c * 8 + i) * NPAD + s * TPS, 8)
        pltpu.sync_copy(accs[i].at[pl.ds(off, TPS)], zbuf)
        pltpu.sync_copy(zbuf, out_h.at[pl.ds(ooff, TPS)])


@functools.partial(
    pl.kernel,
    out_type=_f32((NC * NS * 16,)),
    mesh=_mesh,
    compiler_params=pltpu.CompilerParams(needs_layout_passes=False, use_tc_tiling_on_sc=False),
    scratch_types=dict(
        stab=pltpu.VMEM((NPAD,), jnp.float32),
        srcb=pltpu.VMEM((CH,), jnp.int32),
        dstb=pltpu.VMEM((CH,), jnp.int32),
        obuf=pltpu.VMEM((16,), jnp.float32),
    ),
)
def _f4_masked_sum(src_h, dst_h, s_h, out_h, *, stab, srcb, dstb, obuf):
    c = lax.axis_index("c")
    s = lax.axis_index("s")
    wid = c * NS + s
    base = wid * EPT
    pltpu.sync_copy(s_h, stab)

    def chunk(ci, acc):
        pltpu.sync_copy(src_h.at[pl.ds(pl.multiple_of(base + ci * CH, 128), CH)], srcb)
        pltpu.sync_copy(dst_h.at[pl.ds(pl.multiple_of(base + ci * CH, 128), CH)], dstb)

        def step(j, acc):
            dv = dstb[pl.ds(j * 16, 16)]
            sv = srcb[pl.ds(j * 16, 16)]
            m = dv == 1
            vals = plsc.load_gather(stab, [sv])
            return acc + jnp.where(m, vals, 0.0)

        return lax.fori_loop(0, CH // 16, step, acc)

    acc = lax.fori_loop(0, NCHUNK, chunk, jnp.zeros((16,), jnp.float32))
    obuf[...] = acc
    oof = pl.multiple_of(wid * 16, 8)
    pltpu.sync_copy(obuf, out_h.at[pl.ds(oof, 16)])


def _lrelu(x):
    return jnp.maximum(x, 0.01 * x)


def _t1_body(a1_ref, w0c_ref, b0c_ref, w1t_ref, o_ref):
    a = a1_ref[0:1, :] + a1_ref[1:2, :]
    h = _lrelu(w0c_ref[...] * a + b0c_ref[...])
    o_ref[...] = jnp.dot(w1t_ref[...], h, preferred_element_type=jnp.float32)


def _t2_body(a_ref, bc_ref, wt_ref, o_ref):
    h = _lrelu(a_ref[...] + bc_ref[...])
    o_ref[...] = jnp.dot(wt_ref[...], h, preferred_element_type=jnp.float32)


def _t4_body(p_ref, b3_ref, o_ref):
    o_ref[...] = _lrelu(jnp.sum(p_ref[...]) + b3_ref[0, 0])[None, None]


def kernel(in_feat, edge_index, W0, b0, W1, b1, W2, b2, W3, b3):
    src = edge_index[0].astype(jnp.int32)
    dst = edge_index[1].astype(jnp.int32)
    src_p = jnp.concatenate([src, jnp.zeros((EPAD - E,), jnp.int32)])
    dst_p = jnp.concatenate([dst, jnp.full((EPAD - E,), N, jnp.int32)])
    src2 = src_p.reshape(EPAD // 128, 128)
    dst2 = dst_p.reshape(EPAD // 128, 128)
    x1 = jnp.pad(in_feat[:, 0], (0, NPAD - N))

    a1 = _f1_scalar_scatter(src_p, dst2, x1).reshape(NC, NPAD)

    t1T = pl.pallas_call(_t1_body, out_shape=_f32((16, NPAD)))(
        a1, W0.reshape(16, 1), b0.reshape(16, 1), W1.T)

    a2T = _f_row_scatter(src2, dst2, t1T.reshape(16 * NPAD)).reshape(16, NPAD)

    g2T = pl.pallas_call(_t2_body, out_shape=_f32((16, NPAD)))(
        a2T, b1.reshape(16, 1), W2.T)

    a3T = _f_row_scatter(src2, dst2, g2T.reshape(16 * NPAD)).reshape(16, NPAD)

    sT = pl.pallas_call(_t2_body, out_shape=_f32((1, NPAD)))(
        a3T, b2.reshape(16, 1), W3.T)

    part = _f4_masked_sum(src_p, dst_p, sT.reshape(NPAD)).reshape(NW, 16)

    out = pl.pallas_call(_t4_body, out_shape=_f32((1, 1)))(
        part, b3.reshape(1, 1))
    return out.reshape(1)


# speedup vs baseline: 16.6144x; 1.3016x over previous
import functools

import jax
import jax.numpy as jnp
from jax import lax
from jax.experimental import pallas as pl
from jax.experimental.pallas import tpu as pltpu
from jax.experimental.pallas import tpu_sc as plsc

N = 100000
NPAD = 100096
E = 3200000
EPAD = 3276800
NC, NS = 2, 16
NW = NC * NS
CH = 2048
ROWS = CH // 128
TPS = NPAD // NS
EPT = EPAD // NW
NCHUNK = EPT // CH

_mesh = plsc.VectorSubcoreMesh(core_axis_name="c", subcore_axis_name="s")


def _f32(shape):
    return jax.ShapeDtypeStruct(shape, jnp.float32)


@functools.partial(
    pl.kernel,
    out_type=_f32((NC * NPAD,)),
    mesh=_mesh,
    compiler_params=pltpu.CompilerParams(needs_layout_passes=False, use_tc_tiling_on_sc=False),
    scratch_types=dict(
        xbuf=pltpu.VMEM((NPAD,), jnp.float32),
        srcb=pltpu.VMEM((CH,), jnp.int32),
        dstb=pltpu.VMEM((CH,), jnp.int32),
        valb=pltpu.VMEM((CH,), jnp.float32),
        zbuf=pltpu.VMEM((TPS,), jnp.float32),
        acc=pltpu.VMEM_SHARED((NPAD,), jnp.float32),
    ),
)
def _f1_scalar_scatter(src_h, dst_h, x_h, out_h, *, xbuf, srcb, dstb,
                       valb, zbuf, acc):
    c = lax.axis_index("c")
    s = lax.axis_index("s")
    wid = c * NS + s
    base = wid * EPT
    baserow = base // 128
    def zfill(i, carry):
        zbuf[pl.ds(i * 16, 16)] = jnp.zeros((16,), jnp.float32)
        return carry
    lax.fori_loop(0, TPS // 16, zfill, 0)
    off = pl.multiple_of(s * TPS, 8)
    pltpu.sync_copy(zbuf, acc.at[pl.ds(off, TPS)])
    pltpu.sync_copy(x_h, xbuf)
    plsc.subcore_barrier()

    def chunk(ci, carry):
        eoff = pl.multiple_of(base + ci * CH, 128)
        pltpu.sync_copy(src_h.at[pl.ds(eoff, CH)], srcb)
        pltpu.sync_copy(dst_h.at[pl.ds(eoff, CH)], dstb)
        def fill(j, carry2):
            sv = srcb[pl.ds(j * 16, 16)]
            valb[pl.ds(j * 16, 16)] = plsc.load_gather(xbuf, [sv])
            return carry2
        lax.fori_loop(0, CH // 16, fill, 0)
        pltpu.sync_copy(valb, acc.at[dstb], add=True)
        return carry

    lax.fori_loop(0, NCHUNK, chunk, 0)
    plsc.subcore_barrier()
    off = pl.multiple_of(s * TPS, 8)
    oof = pl.multiple_of(c * NPAD + s * TPS, 8)
    pltpu.sync_copy(acc.at[pl.ds(off, TPS)], zbuf)
    pltpu.sync_copy(zbuf, out_h.at[pl.ds(oof, TPS)])


EPT_B = EPAD // NS
NCHUNK_B = EPT_B // CH

@functools.partial(
    pl.kernel,
    out_type=_f32((NC * 8 * NPAD,)),
    mesh=_mesh,
    compiler_params=pltpu.CompilerParams(needs_layout_passes=False, use_tc_tiling_on_sc=False),
    scratch_types=dict(
        srcb=pltpu.VMEM((CH,), jnp.int32),
        dstb=pltpu.VMEM((CH,), jnp.int32),
        vbs=[pltpu.VMEM((CH,), jnp.float32) for _ in range(8)],
        zbuf=pltpu.VMEM((TPS,), jnp.float32),
        sem=pltpu.SemaphoreType.DMA,
        planes=[pltpu.VMEM_SHARED((NPAD,), jnp.float32) for _ in range(8)],
        accs=[pltpu.VMEM_SHARED((NPAD,), jnp.float32) for _ in range(8)],
    ),
)
def _f_row_scatter(srcf_h, dstf_h, tflat_h, out_h, *, srcb, dstb, vbs, zbuf,
                   sem, planes, accs):
    c = lax.axis_index("c")
    s = lax.axis_index("s")
    base = s * EPT_B
    baserow = base // 128
    off = pl.multiple_of(s * TPS, 8)
    def zfill(i, carry):
        zbuf[pl.ds(i * 16, 16)] = jnp.zeros((16,), jnp.float32)
        return carry
    lax.fori_loop(0, TPS // 16, zfill, 0)
    for i in range(8):
        pltpu.sync_copy(zbuf, accs[i].at[pl.ds(off, TPS)])
    for i in range(8):
        hoff = pl.multiple_of((c * 8 + i) * NPAD + s * TPS, 8)
        pltpu.sync_copy(tflat_h.at[pl.ds(hoff, TPS)], zbuf)
        pltpu.sync_copy(zbuf, planes[i].at[pl.ds(off, TPS)])
    plsc.subcore_barrier()

    def chunk(ci, carry):
        eoff = pl.multiple_of(base + ci * CH, 128)
        pltpu.sync_copy(srcf_h.at[pl.ds(eoff, CH)], srcb)
        pltpu.sync_copy(dstf_h.at[pl.ds(eoff, CH)], dstb)
        gs = [pltpu.async_copy(planes[i].at[srcb], vbs[i], sem)
              for i in range(8)]
        for g in gs:
            g.wait()
        for i in range(8):
            pltpu.sync_copy(vbs[i], accs[i].at[dstb], add=True)
        return carry

    lax.fori_loop(0, NCHUNK_B, chunk, 0)
    plsc.subcore_barrier()
    for i in range(8):
        ooff = pl.multiple_of((c * 8 + i) * NPAD + s * TPS, 8)
        pltpu.sync_copy(accs[i].at[pl.ds(off, TPS)], zbuf)
        pltpu.sync_copy(zbuf, out_h.at[pl.ds(ooff, TPS)])


@functools.partial(
    pl.kernel,
    out_type=_f32((NC * NS * 16,)),
    mesh=_mesh,
    compiler_params=pltpu.CompilerParams(needs_layout_passes=False, use_tc_tiling_on_sc=False),
    scratch_types=dict(
        stab=pltpu.VMEM((NPAD,), jnp.float32),
        srcb=pltpu.VMEM((CH,), jnp.int32),
        dstb=pltpu.VMEM((CH,), jnp.int32),
        obuf=pltpu.VMEM((16,), jnp.float32),
    ),
)
def _f4_masked_sum(src_h, dst_h, s_h, out_h, *, stab, srcb, dstb, obuf):
    c = lax.axis_index("c")
    s = lax.axis_index("s")
    wid = c * NS + s
    base = wid * EPT
    pltpu.sync_copy(s_h, stab)

    def chunk(ci, acc):
        pltpu.sync_copy(src_h.at[pl.ds(pl.multiple_of(base + ci * CH, 128), CH)], srcb)
        pltpu.sync_copy(dst_h.at[pl.ds(pl.multiple_of(base + ci * CH, 128), CH)], dstb)

        def step(j, acc):
            dv = dstb[pl.ds(j * 16, 16)]
            sv = srcb[pl.ds(j * 16, 16)]
            m = dv == 1
            vals = plsc.load_gather(stab, [sv])
            return acc + jnp.where(m, vals, 0.0)

        return lax.fori_loop(0, CH // 16, step, acc)

    acc = lax.fori_loop(0, NCHUNK, chunk, jnp.zeros((16,), jnp.float32))
    obuf[...] = acc
    oof = pl.multiple_of(wid * 16, 8)
    pltpu.sync_copy(obuf, out_h.at[pl.ds(oof, 16)])


def _lrelu(x):
    return jnp.maximum(x, 0.01 * x)


def _t1_body(a1_ref, w0c_ref, b0c_ref, w1t_ref, o_ref):
    a = a1_ref[0:1, :] + a1_ref[1:2, :]
    h = _lrelu(w0c_ref[...] * a + b0c_ref[...])
    o_ref[...] = jnp.dot(w1t_ref[...], h, preferred_element_type=jnp.float32)


def _t2_body(a_ref, bc_ref, wt_ref, o_ref):
    h = _lrelu(a_ref[...] + bc_ref[...])
    o_ref[...] = jnp.dot(wt_ref[...], h, preferred_element_type=jnp.float32)


def _t4_body(p_ref, b3_ref, o_ref):
    o_ref[...] = _lrelu(jnp.sum(p_ref[...]) + b3_ref[0, 0])[None, None]


def kernel(in_feat, edge_index, W0, b0, W1, b1, W2, b2, W3, b3):
    src = edge_index[0].astype(jnp.int32)
    dst = edge_index[1].astype(jnp.int32)
    src_p = jnp.concatenate([src, jnp.zeros((EPAD - E,), jnp.int32)])
    dst_p = jnp.concatenate([dst, jnp.full((EPAD - E,), N, jnp.int32)])
    src2 = src_p.reshape(EPAD // 128, 128)
    dst2 = dst_p.reshape(EPAD // 128, 128)
    x1 = jnp.pad(in_feat[:, 0], (0, NPAD - N))

    a1 = _f1_scalar_scatter(src_p, dst_p, x1).reshape(NC, NPAD)

    t1T = pl.pallas_call(_t1_body, out_shape=_f32((16, NPAD)))(
        a1, W0.reshape(16, 1), b0.reshape(16, 1), W1.T)

    a2T = _f_row_scatter(src_p, dst_p, t1T.reshape(16 * NPAD)).reshape(16, NPAD)

    g2T = pl.pallas_call(_t2_body, out_shape=_f32((16, NPAD)))(
        a2T, b1.reshape(16, 1), W2.T)

    a3T = _f_row_scatter(src_p, dst_p, g2T.reshape(16 * NPAD)).reshape(16, NPAD)

    sT = pl.pallas_call(_t2_body, out_shape=_f32((1, NPAD)))(
        a3T, b2.reshape(16, 1), W3.T)

    part = _f4_masked_sum(src_p, dst_p, sT.reshape(NPAD)).reshape(NW, 16)

    out = pl.pallas_call(_t4_body, out_shape=_f32((1, 1)))(
        part, b3.reshape(1, 1))
    return out.reshape(1)
